# Initial kernel scaffold; baseline (speedup 1.0000x reference)
import jax
import jax.numpy as jnp
from jax.experimental import pallas as pl


def kernel(hidden_states, Wq, bq, Wk, bk, Wv, bv, Wo, bo):
    raise NotImplementedError("write your pallas kernel here")


# speedup vs baseline: 8.2546x; 8.2546x over previous
import jax
import jax.numpy as jnp
from jax.experimental import pallas as pl

S = 2048
D = 1024
H = 16
HD = D // H
K_KEEP = S // 2
ROWS = 512
BLK = 512


def _qkv_body(x_ref, wq_ref, wk_ref, wv_ref, b_ref, q_ref, k_ref, v_ref):
    x = x_ref[...]
    q_ref[0] = jnp.dot(x, wq_ref[0], preferred_element_type=jnp.float32) + b_ref[0, 0:1, :]
    k_ref[0] = jnp.dot(x, wk_ref[0], preferred_element_type=jnp.float32) + b_ref[0, 1:2, :]
    v_ref[0] = jnp.dot(x, wv_ref[0], preferred_element_type=jnp.float32) + b_ref[0, 2:3, :]


def _attn_body(q_ref, k_ref, v_ref, o_ref):
    q = q_ref[0]
    k = k_ref[0]
    s = jax.lax.dot_general(q, k, (((1,), (1,)), ((), ())),
                            preferred_element_type=jnp.float32)
    s = s * jnp.float32(1.0 / 8.0)

    kf = jnp.float32(K_KEEP)
    mask7f = jnp.int32(0x7FFFFFFF)

    def step(i, t):
        bit = jnp.left_shift(jnp.int32(1), 31 - i)
        cand = jnp.bitwise_or(t, bit)
        u = jnp.where(cand < 0, jnp.bitwise_and(cand, mask7f),
                      jnp.bitwise_not(cand))
        tf = jax.lax.bitcast_convert_type(u, jnp.float32)
        cnt = jnp.sum((s >= tf).astype(jnp.float32), axis=1, keepdims=True)
        return jnp.where(cnt >= kf, cand, t)

    t = jax.lax.fori_loop(0, 32, step, jnp.zeros((ROWS, 1), jnp.int32))
    u = jnp.where(t < 0, jnp.bitwise_and(t, mask7f), jnp.bitwise_not(t))
    thr = jax.lax.bitcast_convert_type(u, jnp.float32)

    m = jnp.max(s, axis=1, keepdims=True)
    p = jnp.where(s >= thr, jnp.exp(s - m), jnp.float32(0.0))
    denom = jnp.sum(p, axis=1, keepdims=True)
    ctx = jax.lax.dot_general(p, v_ref[0], (((1,), (0,)), ((), ())),
                              preferred_element_type=jnp.float32)
    o_ref[0] = ctx / denom


def _proj_body(c_ref, wo_ref, bo_ref, o_ref):
    h = pl.program_id(1)

    @pl.when(h == 0)
    def _init():
        o_ref[...] = jnp.broadcast_to(bo_ref[...], o_ref.shape)

    o_ref[...] += jnp.dot(c_ref[0], wo_ref[0],
                          preferred_element_type=jnp.float32)


@jax.jit
def kernel(hidden_states, Wq, bq, Wk, bk, Wv, bv, Wo, bo):
    x = hidden_states.reshape(S, D)
    wq_t = Wq.T.reshape(D, H, HD).transpose(1, 0, 2)
    wk_t = Wk.T.reshape(D, H, HD).transpose(1, 0, 2)
    wv_t = Wv.T.reshape(D, H, HD).transpose(1, 0, 2)
    wo_t = Wo.T.reshape(H, HD, D)
    b_qkv = jnp.stack([bq, bk, bv]).reshape(3, H, HD).transpose(1, 0, 2)

    q, k, v = pl.pallas_call(
        _qkv_body,
        grid=(S // BLK, H),
        in_specs=[
            pl.BlockSpec((BLK, D), lambda r, h: (r, 0)),
            pl.BlockSpec((1, D, HD), lambda r, h: (h, 0, 0)),
            pl.BlockSpec((1, D, HD), lambda r, h: (h, 0, 0)),
            pl.BlockSpec((1, D, HD), lambda r, h: (h, 0, 0)),
            pl.BlockSpec((1, 3, HD), lambda r, h: (h, 0, 0)),
        ],
        out_specs=[
            pl.BlockSpec((1, BLK, HD), lambda r, h: (h, r, 0)),
            pl.BlockSpec((1, BLK, HD), lambda r, h: (h, r, 0)),
            pl.BlockSpec((1, BLK, HD), lambda r, h: (h, r, 0)),
        ],
        out_shape=[jax.ShapeDtypeStruct((H, S, HD), jnp.float32)] * 3,
    )(x, wq_t, wk_t, wv_t, b_qkv)

    ctx = pl.pallas_call(
        _attn_body,
        grid=(H, S // ROWS),
        in_specs=[
            pl.BlockSpec((1, ROWS, HD), lambda h, r: (h, r, 0)),
            pl.BlockSpec((1, S, HD), lambda h, r: (h, 0, 0)),
            pl.BlockSpec((1, S, HD), lambda h, r: (h, 0, 0)),
        ],
        out_specs=pl.BlockSpec((1, ROWS, HD), lambda h, r: (h, r, 0)),
        out_shape=jax.ShapeDtypeStruct((H, S, HD), jnp.float32),
    )(q, k, v)

    out = pl.pallas_call(
        _proj_body,
        grid=(S // BLK, H),
        in_specs=[
            pl.BlockSpec((1, BLK, HD), lambda r, h: (h, r, 0)),
            pl.BlockSpec((1, HD, D), lambda r, h: (h, 0, 0)),
            pl.BlockSpec((1, D), lambda r, h: (0, 0)),
        ],
        out_specs=pl.BlockSpec((BLK, D), lambda r, h: (r, 0)),
        out_shape=jax.ShapeDtypeStruct((S, D), jnp.float32),
    )(ctx, wo_t, bo.reshape(1, D))

    return out.reshape(1, S, D)
